# baseline (speedup 1.0000x reference)
import functools

import jax
import jax.numpy as jnp
from jax import lax
from jax.experimental import pallas as pl
from jax.experimental.pallas import tpu as pltpu
from jax.experimental.pallas import tpu_sc as plsc

_NC = 2
_NS = 16
_LN = 128
_CH = 5


def kernel(x, edges, boundary_loop, boundary_vs, n_verts,
           W0s, W0n, b0, W1s, W1n, b1, Wd, bd, Wo, bo):
    f32, i32 = jnp.float32, jnp.int32
    N, C0 = x.shape
    E = edges.shape[0]
    BL = boundary_vs.shape[0]
    C1 = W0s.shape[1]
    C2 = W1s.shape[1]
    FC = Wd.shape[1]
    O3 = Wo.shape[1]
    NV = O3 // 3
    H = C2 // 2

    ER = -(-E // _LN)
    ERP = -(-ER // (32 * _CH * 2)) * (32 * _CH * 2)
    S16 = -(-(N // _NS + 1) // 8) * 8
    NP = _NS * S16
    BS = BL // _NS

    x4 = jnp.concatenate([x, jnp.zeros((N, 8 - C0), f32)], 1)
    pe = ERP * _LN - E
    epad = jnp.concatenate([jnp.zeros((pe, 1), i32), jnp.full((pe, 1), N, i32)], 1)
    ed = jnp.concatenate([edges.astype(i32), epad], 0).T.reshape(2, ERP, _LN)
    src2 = ed[0]
    dst2 = ed[1]
    vs = boundary_vs.astype(i32)
    bl = boundary_loop.astype(i32)
    z4 = jnp.zeros((S16, 8), f32)
    z16 = jnp.zeros((S16, H), f32)
    W0n4 = jnp.zeros((8, C1), f32).at[:C0].set(W0n)

    mesh = plsc.VectorSubcoreMesh(core_axis_name="c", subcore_axis_name="s")
    scp = pltpu.CompilerParams(use_tc_tiling_on_sc=False)

    def _pipelined_segsum(table, s2_h, d2_h, acc, base, npairs,
                          sbA, dbA, sbB, dbB, rowsA, rowsB,
                          gsemA, gsemB, ssem, D):
        def fire_g(sb, rows, sem):
            return [pltpu.async_copy(table.at[sb.at[j]],
                                     rows.at[pl.ds(j * _LN, _LN), :], sem)
                    for j in range(_CH)]

        def wait_g(sb, rows, sem):
            for j in range(_CH):
                pltpu.make_async_copy(table.at[sb.at[j]],
                                      rows.at[pl.ds(j * _LN, _LN), :],
                                      sem).wait()

        def scat(db, rows):
            sds = [pltpu.async_copy(rows.at[pl.ds(j * _LN, _LN), :],
                                    acc.at[db.at[j]], ssem, add=True)
                   for j in range(_CH)]
            for d_ in sds:
                d_.wait()

        def stage(r0, sb, db):
            pltpu.sync_copy(s2_h.at[pl.ds(r0, _CH), :], sb)
            pltpu.sync_copy(d2_h.at[pl.ds(r0, _CH), :], db)

        def body(i, carry):
            stage(base + i * _CH, sbA, dbA)
            fire_g(sbA, rowsA, gsemA)
            wait_g(sbA, rowsA, gsemA)
            scat(dbA, rowsA)
            return carry

        lax.fori_loop(0, 2 * npairs, body, 0)

    RWA = ERP // (_NC * _NS)
    PAIRS_A = RWA // (2 * _CH)

    @functools.partial(
        pl.kernel,
        out_type=jax.ShapeDtypeStruct((_NC, NP, 8), f32),
        mesh=mesh,
        compiler_params=scp,
        scratch_types=[
            pltpu.VMEM((_CH, _LN), i32), pltpu.VMEM((_CH, _LN), i32),
            pltpu.VMEM((_CH, _LN), i32), pltpu.VMEM((_CH, _LN), i32),
            pltpu.VMEM((_CH * _LN, 8), f32), pltpu.VMEM((_CH * _LN, 8), f32),
            pltpu.VMEM_SHARED((NP, 8), f32),
            pltpu.SemaphoreType.DMA, pltpu.SemaphoreType.DMA,
            pltpu.SemaphoreType.DMA,
        ],
    )
    def seg0(x4_h, s2_h, d2_h, z4_h, aggp_h,
             sbA, dbA, sbB, dbB, rowsA, rowsB, acc, gsemA, gsemB, ssem):
        c = lax.axis_index("c")
        s = lax.axis_index("s")
        w = c * _NS + s
        pltpu.sync_copy(z4_h, acc.at[pl.ds(s * S16, S16), :])
        plsc.subcore_barrier()
        _pipelined_segsum(x4_h, s2_h, d2_h, acc, w * RWA, PAIRS_A,
                          sbA, dbA, sbB, dbB, rowsA, rowsB,
                          gsemA, gsemB, ssem, 4)
        plsc.subcore_barrier()
        pltpu.sync_copy(acc.at[pl.ds(s * S16, S16), :],
                        aggp_h.at[c, pl.ds(s * S16, S16), :])

    aggp = seg0(x4, src2, dst2, z4)

    BM = 2000
    GB = N // BM

    def h1_body(x_r, p0_r, p1_r, ws_r, wn_r, b_r, lo_r, hi_r):
        agg = p0_r[0] + p1_r[0]
        h = (jnp.dot(x_r[...], ws_r[...], preferred_element_type=f32)
             + jnp.dot(agg, wn_r[...], preferred_element_type=f32)
             + b_r[...])
        h = jnp.maximum(h, 0.0)
        lo_r[...] = h[:, :H]
        hi_r[...] = h[:, H:]

    h_lo, h_hi = pl.pallas_call(
        h1_body,
        grid=(GB,),
        in_specs=[
            pl.BlockSpec((BM, C0), lambda i: (i, 0)),
            pl.BlockSpec((1, BM, 8), lambda i: (0, i, 0)),
            pl.BlockSpec((1, BM, 8), lambda i: (1, i, 0)),
            pl.BlockSpec((C0, C1), lambda i: (0, 0)),
            pl.BlockSpec((8, C1), lambda i: (0, 0)),
            pl.BlockSpec((1, C1), lambda i: (0, 0)),
        ],
        out_specs=[pl.BlockSpec((BM, H), lambda i: (i, 0)),
                   pl.BlockSpec((BM, H), lambda i: (i, 0))],
        out_shape=[jax.ShapeDtypeStruct((N, H), f32),
                   jax.ShapeDtypeStruct((N, H), f32)],
    )(x, aggp, aggp, W0s, W0n4, b0.reshape(1, C1))

    RWC = ERP // _NS
    PAIRS_C = RWC // (2 * _CH)
    SFH = S16 * H // _LN

    @functools.partial(
        pl.kernel,
        out_type=(jax.ShapeDtypeStruct((NP, H), f32),
                  jax.ShapeDtypeStruct((NP, H), f32),
                  jax.ShapeDtypeStruct((BL, H), f32),
                  jax.ShapeDtypeStruct((BL, H), f32),
                  jax.ShapeDtypeStruct((BL, H), f32),
                  jax.ShapeDtypeStruct((BL, H), f32),
                  jax.ShapeDtypeStruct((BL, 8), f32)),
        mesh=mesh,
        compiler_params=scp,
        scratch_types=[
            pltpu.VMEM((_CH, _LN), i32), pltpu.VMEM((_CH, _LN), i32),
            pltpu.VMEM((_CH, _LN), i32), pltpu.VMEM((_CH, _LN), i32),
            pltpu.VMEM((_CH * _LN, H), f32), pltpu.VMEM((_CH * _LN, H), f32),
            pltpu.VMEM_SHARED((NP, H), f32),
            pltpu.VMEM((BS,), i32),
            pltpu.VMEM((BS, H), f32), pltpu.VMEM((BS, H), f32),
            pltpu.VMEM((BS, 8), f32),
            pltpu.SemaphoreType.DMA, pltpu.SemaphoreType.DMA,
            pltpu.SemaphoreType.DMA,
        ],
    )
    def seg1(hlo_h, hhi_h, x4_h, s2_h, d2_h, vs_h, bl_h, z16_h,
             agglo_h, agghi_h, hblo_h, hbhi_h, ablo_h, abhi_h, xb_h,
             sbA, dbA, sbB, dbB, rowsA, rowsB, acc, ib, g1, g2, g3,
             gsemA, gsemB, ssem):
        c = lax.axis_index("c")
        s = lax.axis_index("s")
        pltpu.sync_copy(z16_h, acc.at[pl.ds(s * S16, S16), :])
        plsc.subcore_barrier()

        @pl.when(c == 0)
        def _():
            _pipelined_segsum(hlo_h, s2_h, d2_h, acc, s * RWC, PAIRS_C,
                              sbA, dbA, sbB, dbB, rowsA, rowsB,
                              gsemA, gsemB, ssem, H)

        @pl.when(c == 1)
        def _():
            _pipelined_segsum(hhi_h, s2_h, d2_h, acc, s * RWC, PAIRS_C,
                              sbA, dbA, sbB, dbB, rowsA, rowsB,
                              gsemA, gsemB, ssem, H)

        plsc.subcore_barrier()

        @pl.when(c == 0)
        def _():
            pltpu.sync_copy(acc.at[pl.ds(s * S16, S16), :],
                            agglo_h.at[pl.ds(s * S16, S16), :])

        @pl.when(c == 1)
        def _():
            pltpu.sync_copy(acc.at[pl.ds(s * S16, S16), :],
                            agghi_h.at[pl.ds(s * S16, S16), :])

        plsc.subcore_barrier()
        bb = s * BS

        @pl.when(c == 0)
        def _():
            pltpu.sync_copy(vs_h.at[pl.ds(bb, BS)], ib)
            pltpu.async_copy(hlo_h.at[ib], g1, gsemA).wait()
            pltpu.sync_copy(g1, hblo_h.at[pl.ds(bb, BS), :])
            pltpu.async_copy(agglo_h.at[ib], g2, gsemA).wait()
            pltpu.sync_copy(g2, ablo_h.at[pl.ds(bb, BS), :])

        @pl.when(c == 1)
        def _():
            pltpu.sync_copy(vs_h.at[pl.ds(bb, BS)], ib)
            pltpu.async_copy(hhi_h.at[ib], g1, gsemA).wait()
            pltpu.sync_copy(g1, hbhi_h.at[pl.ds(bb, BS), :])
            pltpu.async_copy(agghi_h.at[ib], g2, gsemA).wait()
            pltpu.sync_copy(g2, abhi_h.at[pl.ds(bb, BS), :])
            pltpu.sync_copy(bl_h.at[pl.ds(bb, BS)], ib)
            pltpu.async_copy(x4_h.at[ib], g3, gsemA).wait()
            pltpu.sync_copy(g3, xb_h.at[pl.ds(bb, BS), :])

    (_agg_lo, _agg_hi, hb_lo, hb_hi, ab_lo, ab_hi, xb) = seg1(
        h_lo, h_hi, x4, src2, dst2, vs, bl, z16)

    def tail_body(hbl, hbh, abl, abh, xbr, w1s_r, w1n_r, b1_r,
                  wd_r, bd_r, wo_r, bo_r, out_r):
        hb = jnp.concatenate([hbl[...], hbh[...]], axis=1)
        ab = jnp.concatenate([abl[...], abh[...]], axis=1)
        h2 = jnp.maximum(jnp.dot(hb, w1s_r[...], preferred_element_type=f32)
                         + jnp.dot(ab, w1n_r[...], preferred_element_type=f32)
                         + b1_r[...], 0.0)
        pooled = jnp.mean(h2, axis=0, keepdims=True)
        d_ = jnp.maximum(jnp.dot(pooled, wd_r[...], preferred_element_type=f32)
                         + bd_r[...], 0.0)
        o = jnp.dot(d_, wo_r[...], preferred_element_type=f32) + bo_r[...]
        bm = jnp.mean(xbr[...], axis=0, keepdims=True)
        for r in range(NV):
            out_r[pl.ds(r, 1), :] = o[:, 3 * r:3 * r + 3] + bm[:, :3]

    out = pl.pallas_call(
        tail_body,
        out_shape=jax.ShapeDtypeStruct((NV, 3), f32),
    )(hb_lo, hb_hi, ab_lo, ab_hi, xb, W1s, W1n, b1.reshape(1, C2),
      Wd, bd.reshape(1, FC), Wo, bo.reshape(1, O3))
    return out


# speedup vs baseline: 1.5000x; 1.5000x over previous
import functools

import jax
import jax.numpy as jnp
from jax import lax
from jax.experimental import pallas as pl
from jax.experimental.pallas import tpu as pltpu
from jax.experimental.pallas import tpu_sc as plsc

_NC = 2
_NS = 16
_LN = 128
_CH = 8


def kernel(x, edges, boundary_loop, boundary_vs, n_verts,
           W0s, W0n, b0, W1s, W1n, b1, Wd, bd, Wo, bo):
    f32, i32 = jnp.float32, jnp.int32
    N, C0 = x.shape
    E = edges.shape[0]
    BL = boundary_vs.shape[0]
    C1 = W0s.shape[1]
    C2 = W1s.shape[1]
    FC = Wd.shape[1]
    O3 = Wo.shape[1]
    NV = O3 // 3
    H = C2 // 2

    ER = -(-E // _LN)
    ERP = -(-ER // (32 * _CH)) * (32 * _CH)
    S16 = -(-(N // _NS + 1) // 8) * 8
    NP = _NS * S16
    BS = BL // _NS

    x4 = jnp.concatenate([x, jnp.zeros((N, 8 - C0), f32)], 1)
    pe = ERP * _LN - E
    epad = jnp.concatenate([jnp.zeros((pe, 1), i32), jnp.full((pe, 1), N, i32)], 1)
    ed = jnp.concatenate([edges.astype(i32), epad], 0).T.reshape(2, ERP, _LN)
    src2 = ed[0]
    dst2 = ed[1]
    vs = boundary_vs.astype(i32)
    bl = boundary_loop.astype(i32)
    z4 = jnp.zeros((S16, 8), f32)
    z16 = jnp.zeros((S16, H), f32)
    W0n4 = jnp.zeros((8, C1), f32).at[:C0].set(W0n)

    mesh = plsc.VectorSubcoreMesh(core_axis_name="c", subcore_axis_name="s")
    scp = pltpu.CompilerParams(use_tc_tiling_on_sc=False)

    def _segsum_loop(table, s2_h, d2_h, acc, base, nchunks,
                     sbA, dbA, rowsA, gsemA, ssem):
        def fire_g(sb, rows, sem):
            return [pltpu.async_copy(table.at[sb.at[j]],
                                     rows.at[pl.ds(j * _LN, _LN), :], sem)
                    for j in range(_CH)]

        def scat(db, rows):
            sds = [pltpu.async_copy(rows.at[pl.ds(j * _LN, _LN), :],
                                    acc.at[db.at[j]], ssem, add=True)
                   for j in range(_CH)]
            for d_ in sds:
                d_.wait()

        def stage(r0, sb, db):
            pltpu.sync_copy(s2_h.at[pl.ds(r0, _CH), :], sb)
            pltpu.sync_copy(d2_h.at[pl.ds(r0, _CH), :], db)

        def body(i, carry):
            stage(base + i * _CH, sbA, dbA)
            gds = fire_g(sbA, rowsA, gsemA)
            for d_ in gds:
                d_.wait()
            scat(dbA, rowsA)
            return carry

        lax.fori_loop(0, nchunks, body, 0)

    RWA = ERP // (_NC * _NS)
    NCH_A = RWA // _CH

    @functools.partial(
        pl.kernel,
        out_type=jax.ShapeDtypeStruct((_NC, NP, 8), f32),
        mesh=mesh,
        compiler_params=scp,
        scratch_types=[
            pltpu.VMEM((_CH, _LN), i32), pltpu.VMEM((_CH, _LN), i32),
            pltpu.VMEM((_CH * _LN, 8), f32),
            pltpu.VMEM_SHARED((NP, 8), f32),
            pltpu.SemaphoreType.DMA, pltpu.SemaphoreType.DMA,
        ],
    )
    def seg0(x4_h, s2_h, d2_h, z4_h, aggp_h,
             sbA, dbA, rowsA, acc, gsemA, ssem):
        c = lax.axis_index("c")
        s = lax.axis_index("s")
        w = c * _NS + s
        pltpu.sync_copy(z4_h, acc.at[pl.ds(s * S16, S16), :])
        plsc.subcore_barrier()
        _segsum_loop(x4_h, s2_h, d2_h, acc, w * RWA, NCH_A,
                     sbA, dbA, rowsA, gsemA, ssem)
        plsc.subcore_barrier()
        pltpu.sync_copy(acc.at[pl.ds(s * S16, S16), :],
                        aggp_h.at[c, pl.ds(s * S16, S16), :])

    aggp = seg0(x4, src2, dst2, z4)

    BM = 2000
    GB = N // BM

    def h1_body(x_r, p0_r, p1_r, ws_r, wn_r, b_r, lo_r, hi_r):
        agg = p0_r[0] + p1_r[0]
        h = (jnp.dot(x_r[...], ws_r[...], preferred_element_type=f32)
             + jnp.dot(agg, wn_r[...], preferred_element_type=f32)
             + b_r[...])
        h = jnp.maximum(h, 0.0)
        lo_r[...] = h[:, :H]
        hi_r[...] = h[:, H:]

    h_lo, h_hi = pl.pallas_call(
        h1_body,
        grid=(GB,),
        in_specs=[
            pl.BlockSpec((BM, C0), lambda i: (i, 0)),
            pl.BlockSpec((1, BM, 8), lambda i: (0, i, 0)),
            pl.BlockSpec((1, BM, 8), lambda i: (1, i, 0)),
            pl.BlockSpec((C0, C1), lambda i: (0, 0)),
            pl.BlockSpec((8, C1), lambda i: (0, 0)),
            pl.BlockSpec((1, C1), lambda i: (0, 0)),
        ],
        out_specs=[pl.BlockSpec((BM, H), lambda i: (i, 0)),
                   pl.BlockSpec((BM, H), lambda i: (i, 0))],
        out_shape=[jax.ShapeDtypeStruct((N, H), f32),
                   jax.ShapeDtypeStruct((N, H), f32)],
    )(x, aggp, aggp, W0s, W0n4, b0.reshape(1, C1))

    RWC = ERP // _NS
    NCH_C = RWC // _CH
    SFH = S16 * H // _LN

    @functools.partial(
        pl.kernel,
        out_type=(jax.ShapeDtypeStruct((NP, H), f32),
                  jax.ShapeDtypeStruct((NP, H), f32),
                  jax.ShapeDtypeStruct((BL, H), f32),
                  jax.ShapeDtypeStruct((BL, H), f32),
                  jax.ShapeDtypeStruct((BL, H), f32),
                  jax.ShapeDtypeStruct((BL, H), f32),
                  jax.ShapeDtypeStruct((BL, 8), f32)),
        mesh=mesh,
        compiler_params=scp,
        scratch_types=[
            pltpu.VMEM((_CH, _LN), i32), pltpu.VMEM((_CH, _LN), i32),
            pltpu.VMEM((_CH * _LN, H), f32),
            pltpu.VMEM_SHARED((NP, H), f32),
            pltpu.VMEM((BS,), i32),
            pltpu.VMEM((BS, H), f32), pltpu.VMEM((BS, H), f32),
            pltpu.VMEM((BS, 8), f32),
            pltpu.SemaphoreType.DMA, pltpu.SemaphoreType.DMA,
        ],
    )
    def seg1(hlo_h, hhi_h, x4_h, s2_h, d2_h, vs_h, bl_h, z16_h,
             agglo_h, agghi_h, hblo_h, hbhi_h, ablo_h, abhi_h, xb_h,
             sbA, dbA, rowsA, acc, ib, g1, g2, g3,
             gsemA, ssem):
        c = lax.axis_index("c")
        s = lax.axis_index("s")
        pltpu.sync_copy(z16_h, acc.at[pl.ds(s * S16, S16), :])
        plsc.subcore_barrier()

        @pl.when(c == 0)
        def _():
            _segsum_loop(hlo_h, s2_h, d2_h, acc, s * RWC, NCH_C,
                         sbA, dbA, rowsA, gsemA, ssem)

        @pl.when(c == 1)
        def _():
            _segsum_loop(hhi_h, s2_h, d2_h, acc, s * RWC, NCH_C,
                         sbA, dbA, rowsA, gsemA, ssem)

        plsc.subcore_barrier()

        @pl.when(c == 0)
        def _():
            pltpu.sync_copy(acc.at[pl.ds(s * S16, S16), :],
                            agglo_h.at[pl.ds(s * S16, S16), :])

        @pl.when(c == 1)
        def _():
            pltpu.sync_copy(acc.at[pl.ds(s * S16, S16), :],
                            agghi_h.at[pl.ds(s * S16, S16), :])

        plsc.subcore_barrier()
        bb = s * BS

        @pl.when(c == 0)
        def _():
            pltpu.sync_copy(vs_h.at[pl.ds(bb, BS)], ib)
            pltpu.async_copy(hlo_h.at[ib], g1, gsemA).wait()
            pltpu.sync_copy(g1, hblo_h.at[pl.ds(bb, BS), :])
            pltpu.async_copy(agglo_h.at[ib], g2, gsemA).wait()
            pltpu.sync_copy(g2, ablo_h.at[pl.ds(bb, BS), :])

        @pl.when(c == 1)
        def _():
            pltpu.sync_copy(vs_h.at[pl.ds(bb, BS)], ib)
            pltpu.async_copy(hhi_h.at[ib], g1, gsemA).wait()
            pltpu.sync_copy(g1, hbhi_h.at[pl.ds(bb, BS), :])
            pltpu.async_copy(agghi_h.at[ib], g2, gsemA).wait()
            pltpu.sync_copy(g2, abhi_h.at[pl.ds(bb, BS), :])
            pltpu.sync_copy(bl_h.at[pl.ds(bb, BS)], ib)
            pltpu.async_copy(x4_h.at[ib], g3, gsemA).wait()
            pltpu.sync_copy(g3, xb_h.at[pl.ds(bb, BS), :])

    (_agg_lo, _agg_hi, hb_lo, hb_hi, ab_lo, ab_hi, xb) = seg1(
        h_lo, h_hi, x4, src2, dst2, vs, bl, z16)

    def tail_body(hbl, hbh, abl, abh, xbr, w1s_r, w1n_r, b1_r,
                  wd_r, bd_r, wo_r, bo_r, out_r):
        hb = jnp.concatenate([hbl[...], hbh[...]], axis=1)
        ab = jnp.concatenate([abl[...], abh[...]], axis=1)
        h2 = jnp.maximum(jnp.dot(hb, w1s_r[...], preferred_element_type=f32)
                         + jnp.dot(ab, w1n_r[...], preferred_element_type=f32)
                         + b1_r[...], 0.0)
        pooled = jnp.mean(h2, axis=0, keepdims=True)
        d_ = jnp.maximum(jnp.dot(pooled, wd_r[...], preferred_element_type=f32)
                         + bd_r[...], 0.0)
        o = jnp.dot(d_, wo_r[...], preferred_element_type=f32) + bo_r[...]
        bm = jnp.mean(xbr[...], axis=0, keepdims=True)
        for r in range(NV):
            out_r[pl.ds(r, 1), :] = o[:, 3 * r:3 * r + 3] + bm[:, :3]

    out = pl.pallas_call(
        tail_body,
        out_shape=jax.ShapeDtypeStruct((NV, 3), f32),
    )(hb_lo, hb_hi, ab_lo, ab_hi, xb, W1s, W1n, b1.reshape(1, C2),
      Wd, bd.reshape(1, FC), Wo, bo.reshape(1, O3))
    return out
